# baseline (speedup 1.0000x reference)
import functools

import jax
import jax.numpy as jnp
from jax import lax
from jax.experimental import pallas as pl
from jax.experimental.pallas import tpu as pltpu
from jax.experimental.pallas import tpu_sc as plsc

N = 10000
E = 320000
D = 128
H = 128
NP = 10112

NC = 2
NS = 16
NW = NC * NS
CH = 64
NCHUNK = 160
WIN = 16
NWIN = NCHUNK // WIN
EPT = NCHUNK * CH
EPAD = NW * EPT
PAD_NODE = 10016
ROWS_PT = NP // NS

_sc_mesh = plsc.VectorSubcoreMesh(core_axis_name="c", subcore_axis_name="s")


def _zero_vmem(ref, nwords):
    z = jnp.zeros((16,), jnp.float32)

    def body(i, _):
        ref[pl.ds(i * 16, 16)] = z
        return 0

    lax.fori_loop(0, nwords // 16, body, 0)


@functools.partial(
    pl.kernel,
    out_type=jax.ShapeDtypeStruct((NC, 2, NP), jnp.float32),
    mesh=_sc_mesh,
    scratch_types=[
        pltpu.VMEM((NCHUNK, CH), jnp.int32),
        pltpu.VMEM((NCHUNK, CH), jnp.int32),
        pltpu.VMEM((CH,), jnp.float32),
        pltpu.VMEM((640,), jnp.float32),
        pltpu.VMEM_SHARED((NP,), jnp.float32),
        pltpu.VMEM_SHARED((NP,), jnp.float32),
    ],
)
def _degrees_sc(srcp_hbm, dstp_hbm, out_hbm, src_v, dst_v, ones_v, zer_v,
                hsrc_sh, hdst_sh):
    c = lax.axis_index("c")
    s = lax.axis_index("s")
    w = s * NC + c

    one = jnp.ones((16,), jnp.float32)

    def initb(i, _):
        ones_v[pl.ds(i * 16, 16)] = one
        return 0

    lax.fori_loop(0, CH // 16, initb, 0)
    _zero_vmem(zer_v, 640)

    pltpu.sync_copy(zer_v.at[pl.ds(0, ROWS_PT)], hsrc_sh.at[pl.ds(s * ROWS_PT, ROWS_PT)])
    pltpu.sync_copy(zer_v.at[pl.ds(0, ROWS_PT)], hdst_sh.at[pl.ds(s * ROWS_PT, ROWS_PT)])

    pltpu.sync_copy(srcp_hbm.at[w], src_v)
    pltpu.sync_copy(dstp_hbm.at[w], dst_v)

    plsc.subcore_barrier()

    def hist_body(j, _):
        pltpu.sync_copy(ones_v, hsrc_sh.at[src_v.at[j]], add=True)
        pltpu.sync_copy(ones_v, hdst_sh.at[dst_v.at[j]], add=True)
        return 0

    lax.fori_loop(0, NCHUNK, hist_body, 0)

    plsc.subcore_barrier()

    @pl.when(s == 0)
    def _():
        pltpu.sync_copy(hsrc_sh, out_hbm.at[c, 0, :])
        pltpu.sync_copy(hdst_sh, out_hbm.at[c, 1, :])


@functools.partial(
    pl.kernel,
    out_type=jax.ShapeDtypeStruct((NC, NP, H), jnp.float32),
    mesh=_sc_mesh,
    scratch_types=[
        pltpu.VMEM((WIN, CH), jnp.int32),
        pltpu.VMEM((WIN, CH), jnp.int32),
        pltpu.VMEM((WIN, CH), jnp.int32),
        pltpu.VMEM((WIN, CH), jnp.int32),
        pltpu.VMEM((CH, H), jnp.float32),
        pltpu.VMEM((CH, H), jnp.float32),
        pltpu.VMEM_SHARED((NP, H), jnp.float32),
        pltpu.SemaphoreType.DMA,
        pltpu.SemaphoreType.DMA,
        pltpu.SemaphoreType.DMA,
        pltpu.SemaphoreType.DMA,
        pltpu.SemaphoreType.DMA,
    ],
)
def _aggregate_sc(hs_hbm, srcp_hbm, dstp_hbm, out_hbm, swin0, swin1, dwin0,
                  dwin1, rows0, rows1, agg_sh, wsem, gsem0, gsem1, ssem0,
                  ssem1):
    c = lax.axis_index("c")
    s = lax.axis_index("s")
    w = s * NC + c
    swin = (swin0, swin1)
    dwin = (dwin0, dwin1)
    rows = (rows0, rows1)
    gsem = (gsem0, gsem1)
    ssem = (ssem0, ssem1)

    wd = [
        pltpu.async_copy(srcp_hbm.at[w, pl.ds(0, WIN), :], swin0, wsem),
        pltpu.async_copy(dstp_hbm.at[w, pl.ds(0, WIN), :], dwin0, wsem),
    ]

    z = jnp.zeros((16,), jnp.float32)

    def zrow(i, _):
        rows0[i // (H // 16), pl.ds((i % (H // 16)) * 16, 16)] = z
        return 0

    lax.fori_loop(0, CH * H // 16, zrow, 0)

    base = s * ROWS_PT

    def zbody(i, _):
        pltpu.sync_copy(rows0, agg_sh.at[pl.ds(base + i * CH, CH), :])
        return 0

    lax.fori_loop(0, ROWS_PT // CH, zbody, 0)
    rem = ROWS_PT % CH
    if rem:
        pltpu.sync_copy(
            rows0.at[pl.ds(0, rem), :],
            agg_sh.at[pl.ds(base + (ROWS_PT // CH) * CH, rem), :],
        )

    plsc.subcore_barrier()

    wdesc = {0: wd}
    gd = {}
    sprev = {}
    for p in range(NCHUNK + 1):
        if p < NCHUNK:
            g, r = p // WIN, p % WIN
            b = p % 2
            if r == 0:
                for d in wdesc[g]:
                    d.wait()
            if b in sprev:
                sprev.pop(b).wait()
            if r == 1 and g + 1 < NWIN:
                nb = (g + 1) % 2
                off = (g + 1) * WIN
                wdesc[g + 1] = [
                    pltpu.async_copy(srcp_hbm.at[w, pl.ds(off, WIN), :],
                                     swin[nb], wsem),
                    pltpu.async_copy(dstp_hbm.at[w, pl.ds(off, WIN), :],
                                     dwin[nb], wsem),
                ]
            gd[b] = (
                pltpu.async_copy(hs_hbm.at[swin[g % 2].at[r]], rows[b],
                                 gsem[b]),
                dwin[g % 2],
                r,
            )
        ob = (p - 1) % 2
        if p >= 1:
            d, dwref, r_ = gd.pop(ob)
            d.wait()
            sprev[ob] = pltpu.async_copy(rows[ob], agg_sh.at[dwref.at[r_]],
                                         ssem[ob], add=True)
    sprev[0].wait()
    sprev[1].wait()

    plsc.subcore_barrier()

    sl = pl.ds(s * ROWS_PT, ROWS_PT)
    pltpu.sync_copy(agg_sh.at[sl, :], out_hbm.at[c, sl, :])


def _norms_body(degp_ref, x_ref, hs_ref, ns_ref, nd_ref):
    d_src = degp_ref[0, 0] + degp_ref[1, 0]
    d_dst = degp_ref[0, 1] + degp_ref[1, 1]
    ns = jax.lax.rsqrt(jnp.maximum(d_src, 1.0))
    nd = jax.lax.rsqrt(jnp.maximum(d_dst, 1.0))
    ns_ref[...] = ns
    nd_ref[...] = nd
    hs_ref[...] = x_ref[...] * ns


def _norms_scale(degp, x_pad):
    RB = 1264
    grid = (NP // RB,)
    return pl.pallas_call(
        _norms_body,
        grid=grid,
        in_specs=[
            pl.BlockSpec((NC, 2, RB, 1), lambda i: (0, 0, i, 0)),
            pl.BlockSpec((RB, D), lambda i: (i, 0)),
        ],
        out_specs=[
            pl.BlockSpec((RB, D), lambda i: (i, 0)),
            pl.BlockSpec((RB, 1), lambda i: (i, 0)),
            pl.BlockSpec((RB, 1), lambda i: (i, 0)),
        ],
        out_shape=[
            jax.ShapeDtypeStruct((NP, D), jnp.float32),
            jax.ShapeDtypeStruct((NP, 1), jnp.float32),
            jax.ShapeDtypeStruct((NP, 1), jnp.float32),
        ],
    )(degp, x_pad)


def _layer_body(aggp_ref, ns_ref, nd_ref, w_ref, b_ref, out_ref, *, scale_out):
    a = (aggp_ref[0] + aggp_ref[1]) * nd_ref[...]
    h = jnp.dot(a, w_ref[...], preferred_element_type=jnp.float32) + b_ref[...]
    h = jnp.maximum(h, 0.0)
    if scale_out:
        h = h * ns_ref[...]
    out_ref[...] = h


def _layer_end(aggp, ns, nd, w, b, scale_out):
    RB = 1264
    grid = (NP // RB,)
    return pl.pallas_call(
        functools.partial(_layer_body, scale_out=scale_out),
        grid=grid,
        in_specs=[
            pl.BlockSpec((NC, RB, H), lambda i: (0, i, 0)),
            pl.BlockSpec((RB, 1), lambda i: (i, 0)),
            pl.BlockSpec((RB, 1), lambda i: (i, 0)),
            pl.BlockSpec((H, H), lambda i: (0, 0)),
            pl.BlockSpec((1, H), lambda i: (0, 0)),
        ],
        out_specs=pl.BlockSpec((RB, H), lambda i: (i, 0)),
        out_shape=jax.ShapeDtypeStruct((NP, H), jnp.float32),
    )(aggp, ns, nd, w, b.reshape(1, H))


def _head_body(xr_ref, wd_ref, bd_ref, out_ref):
    out_ref[...] = (
        jnp.dot(xr_ref[...], wd_ref[...], preferred_element_type=jnp.float32)
        + bd_ref[0, 0]
    )


def _head(xr, wd, bd):
    return pl.pallas_call(
        _head_body,
        out_shape=jax.ShapeDtypeStruct((N // 4, 1), jnp.float32),
    )(xr, wd, bd.reshape(1, 1))


def kernel(x, edge_index, W1, b1, W2, b2, Wd, bd):
    pad = (jnp.arange(EPAD - E, dtype=jnp.int32) % (NP - N)) + N
    srcp = jnp.concatenate([edge_index[0], pad]).reshape(NW, NCHUNK, CH)
    dstp = jnp.concatenate([edge_index[1], pad]).reshape(NW, NCHUNK, CH)
    x_pad = jnp.zeros((NP, D), x.dtype).at[:N].set(x)

    degp = _degrees_sc(srcp, dstp).reshape(NC, 2, NP, 1)
    hs1, ns, nd = _norms_scale(degp, x_pad)

    aggp1 = _aggregate_sc(hs1, srcp, dstp)
    hs2 = _layer_end(aggp1, ns, nd, W1, b1, scale_out=True)

    aggp2 = _aggregate_sc(hs2, srcp, dstp)
    h2 = _layer_end(aggp2, ns, nd, W2, b2, scale_out=False)

    xr = h2[:N].reshape(N // 4, 4 * H)
    return _head(xr, Wd, bd)


# speedup vs baseline: 1.1283x; 1.1283x over previous
import functools

import jax
import jax.numpy as jnp
from jax import lax
from jax.experimental import pallas as pl
from jax.experimental.pallas import tpu as pltpu
from jax.experimental.pallas import tpu_sc as plsc

N = 10000
E = 320000
D = 128
H = 128
NP = 10112

NC = 2
NS = 16
NW = NC * NS
CH = 128
NCHUNK = 80
WIN = 16
NWIN = NCHUNK // WIN
EPT = NCHUNK * CH
EPAD = NW * EPT
PAD_NODE = 10016
ROWS_PT = NP // NS

_sc_mesh = plsc.VectorSubcoreMesh(core_axis_name="c", subcore_axis_name="s")


def _zero_vmem(ref, nwords):
    z = jnp.zeros((16,), jnp.float32)

    def body(i, _):
        ref[pl.ds(i * 16, 16)] = z
        return 0

    lax.fori_loop(0, nwords // 16, body, 0)


@functools.partial(
    pl.kernel,
    out_type=jax.ShapeDtypeStruct((NC, 2 * NP), jnp.float32),
    mesh=_sc_mesh,
    scratch_types=[
        pltpu.VMEM((2 * NCHUNK, CH), jnp.int32),
        pltpu.VMEM((CH,), jnp.float32),
        pltpu.VMEM((1280,), jnp.float32),
        pltpu.VMEM_SHARED((2 * NP,), jnp.float32),
        pltpu.SemaphoreType.DMA,
    ],
)
def _degrees_sc(srcp_hbm, dstp_hbm, out_hbm, idx_v, ones_v, zer_v, hist_sh,
                hsem):
    c = lax.axis_index("c")
    s = lax.axis_index("s")
    w = s * NC + c

    sd = pltpu.async_copy(srcp_hbm.at[w], idx_v.at[pl.ds(0, NCHUNK), :], hsem)
    dd = pltpu.async_copy(dstp_hbm.at[w], idx_v.at[pl.ds(NCHUNK, NCHUNK), :],
                          hsem)

    one = jnp.ones((16,), jnp.float32)

    def initb(i, _):
        ones_v[pl.ds(i * 16, 16)] = one
        return 0

    lax.fori_loop(0, CH // 16, initb, 0)
    _zero_vmem(zer_v, 1280)

    pltpu.sync_copy(zer_v.at[pl.ds(0, 2 * ROWS_PT)],
                    hist_sh.at[pl.ds(s * 2 * ROWS_PT, 2 * ROWS_PT)])
    sd.wait()
    dd.wait()

    npv = jnp.full((16,), NP, jnp.int32)

    def offs(i, _):
        r = NCHUNK + i // (CH // 16)
        k = (i % (CH // 16)) * 16
        idx_v[r, pl.ds(k, 16)] = idx_v[r, pl.ds(k, 16)] + npv
        return 0

    lax.fori_loop(0, NCHUNK * (CH // 16), offs, 0)

    plsc.subcore_barrier()

    descs = [
        pltpu.async_copy(ones_v, hist_sh.at[idx_v.at[j]], hsem, add=True)
        for j in range(2 * NCHUNK)
    ]
    for d in descs:
        d.wait()

    plsc.subcore_barrier()

    @pl.when(s == 0)
    def _():
        pltpu.sync_copy(hist_sh, out_hbm.at[c, :])


@functools.partial(
    pl.kernel,
    out_type=jax.ShapeDtypeStruct((NC, NP, H), jnp.float32),
    mesh=_sc_mesh,
    scratch_types=[
        pltpu.VMEM((WIN, CH), jnp.int32),
        pltpu.VMEM((WIN, CH), jnp.int32),
        pltpu.VMEM((WIN, CH), jnp.int32),
        pltpu.VMEM((WIN, CH), jnp.int32),
        pltpu.VMEM((CH, H), jnp.float32),
        pltpu.VMEM((CH, H), jnp.float32),
        pltpu.VMEM_SHARED((NP, H), jnp.float32),
        pltpu.SemaphoreType.DMA,
        pltpu.SemaphoreType.DMA,
        pltpu.SemaphoreType.DMA,
        pltpu.SemaphoreType.DMA,
        pltpu.SemaphoreType.DMA,
    ],
)
def _aggregate_sc(hs_hbm, srcp_hbm, dstp_hbm, out_hbm, swin0, swin1, dwin0,
                  dwin1, rows0, rows1, agg_sh, wsem, gsem0, gsem1, ssem0,
                  ssem1):
    c = lax.axis_index("c")
    s = lax.axis_index("s")
    w = s * NC + c
    swin = (swin0, swin1)
    dwin = (dwin0, dwin1)
    rows = (rows0, rows1)
    gsem = (gsem0, gsem1)
    ssem = (ssem0, ssem1)

    wd = [
        pltpu.async_copy(srcp_hbm.at[w, pl.ds(0, WIN), :], swin0, wsem),
        pltpu.async_copy(dstp_hbm.at[w, pl.ds(0, WIN), :], dwin0, wsem),
    ]

    z = jnp.zeros((16,), jnp.float32)

    def zrow(i, _):
        rows0[i // (H // 16), pl.ds((i % (H // 16)) * 16, 16)] = z
        return 0

    lax.fori_loop(0, CH * H // 16, zrow, 0)

    base = s * ROWS_PT

    def zbody(i, _):
        pltpu.sync_copy(rows0, agg_sh.at[pl.ds(base + i * CH, CH), :])
        return 0

    lax.fori_loop(0, ROWS_PT // CH, zbody, 0)
    rem = ROWS_PT % CH
    if rem:
        pltpu.sync_copy(
            rows0.at[pl.ds(0, rem), :],
            agg_sh.at[pl.ds(base + (ROWS_PT // CH) * CH, rem), :],
        )

    plsc.subcore_barrier()

    wdesc = {0: wd}
    gd = {}
    sprev = {}
    for p in range(NCHUNK + 1):
        if p < NCHUNK:
            g, r = p // WIN, p % WIN
            b = p % 2
            if r == 0:
                for d in wdesc[g]:
                    d.wait()
            if b in sprev:
                sprev.pop(b).wait()
            if r == 1 and g + 1 < NWIN:
                nb = (g + 1) % 2
                off = (g + 1) * WIN
                wdesc[g + 1] = [
                    pltpu.async_copy(srcp_hbm.at[w, pl.ds(off, WIN), :],
                                     swin[nb], wsem),
                    pltpu.async_copy(dstp_hbm.at[w, pl.ds(off, WIN), :],
                                     dwin[nb], wsem),
                ]
            gd[b] = (
                pltpu.async_copy(hs_hbm.at[swin[g % 2].at[r]], rows[b],
                                 gsem[b]),
                dwin[g % 2],
                r,
            )
        ob = (p - 1) % 2
        if p >= 1:
            d, dwref, r_ = gd.pop(ob)
            d.wait()
            sprev[ob] = pltpu.async_copy(rows[ob], agg_sh.at[dwref.at[r_]],
                                         ssem[ob], add=True)
    sprev[0].wait()
    sprev[1].wait()

    plsc.subcore_barrier()

    sl = pl.ds(s * ROWS_PT, ROWS_PT)
    pltpu.sync_copy(agg_sh.at[sl, :], out_hbm.at[c, sl, :])


def _norms_body(degp_ref, x_ref, hs_ref, ns_ref, nd_ref):
    d_src = degp_ref[0, 0] + degp_ref[1, 0]
    d_dst = degp_ref[0, 1] + degp_ref[1, 1]
    ns = jax.lax.rsqrt(jnp.maximum(d_src, 1.0))
    nd = jax.lax.rsqrt(jnp.maximum(d_dst, 1.0))
    ns_ref[...] = ns
    nd_ref[...] = nd
    hs_ref[...] = x_ref[...] * ns


def _norms_scale(degp, x_pad):
    RB = 1264
    grid = (NP // RB,)
    return pl.pallas_call(
        _norms_body,
        grid=grid,
        in_specs=[
            pl.BlockSpec((NC, 2, RB, 1), lambda i: (0, 0, i, 0)),
            pl.BlockSpec((RB, D), lambda i: (i, 0)),
        ],
        out_specs=[
            pl.BlockSpec((RB, D), lambda i: (i, 0)),
            pl.BlockSpec((RB, 1), lambda i: (i, 0)),
            pl.BlockSpec((RB, 1), lambda i: (i, 0)),
        ],
        out_shape=[
            jax.ShapeDtypeStruct((NP, D), jnp.float32),
            jax.ShapeDtypeStruct((NP, 1), jnp.float32),
            jax.ShapeDtypeStruct((NP, 1), jnp.float32),
        ],
    )(degp, x_pad)


def _layer_body(aggp_ref, ns_ref, nd_ref, w_ref, b_ref, out_ref, *, scale_out):
    a = (aggp_ref[0] + aggp_ref[1]) * nd_ref[...]
    h = jnp.dot(a, w_ref[...], preferred_element_type=jnp.float32) + b_ref[...]
    h = jnp.maximum(h, 0.0)
    if scale_out:
        h = h * ns_ref[...]
    out_ref[...] = h


def _layer_end(aggp, ns, nd, w, b, scale_out):
    RB = 1264
    grid = (NP // RB,)
    return pl.pallas_call(
        functools.partial(_layer_body, scale_out=scale_out),
        grid=grid,
        in_specs=[
            pl.BlockSpec((NC, RB, H), lambda i: (0, i, 0)),
            pl.BlockSpec((RB, 1), lambda i: (i, 0)),
            pl.BlockSpec((RB, 1), lambda i: (i, 0)),
            pl.BlockSpec((H, H), lambda i: (0, 0)),
            pl.BlockSpec((1, H), lambda i: (0, 0)),
        ],
        out_specs=pl.BlockSpec((RB, H), lambda i: (i, 0)),
        out_shape=jax.ShapeDtypeStruct((NP, H), jnp.float32),
    )(aggp, ns, nd, w, b.reshape(1, H))


def _head_body(xr_ref, wd_ref, bd_ref, out_ref):
    out_ref[...] = (
        jnp.dot(xr_ref[...], wd_ref[...], preferred_element_type=jnp.float32)
        + bd_ref[0, 0]
    )


def _head(xr, wd, bd):
    return pl.pallas_call(
        _head_body,
        out_shape=jax.ShapeDtypeStruct((N // 4, 1), jnp.float32),
    )(xr, wd, bd.reshape(1, 1))


def kernel(x, edge_index, W1, b1, W2, b2, Wd, bd):
    pad = (jnp.arange(EPAD - E, dtype=jnp.int32) % (NP - N)) + N
    srcp = jnp.concatenate([edge_index[0], pad]).reshape(NW, NCHUNK, CH)
    dstp = jnp.concatenate([edge_index[1], pad]).reshape(NW, NCHUNK, CH)
    x_pad = jnp.zeros((NP, D), x.dtype).at[:N].set(x)

    degp = _degrees_sc(srcp, dstp).reshape(NC, 2, NP, 1)

    hs1, ns, nd = _norms_scale(degp, x_pad)

    aggp1 = _aggregate_sc(hs1, srcp, dstp)
    hs2 = _layer_end(aggp1, ns, nd, W1, b1, scale_out=True)

    aggp2 = _aggregate_sc(hs2, srcp, dstp)
    h2 = _layer_end(aggp2, ns, nd, W2, b2, scale_out=False)

    xr = h2[:N].reshape(N // 4, 4 * H)
    return _head(xr, Wd, bd)


# speedup vs baseline: 1.1658x; 1.0333x over previous
import functools

import jax
import jax.numpy as jnp
from jax import lax
from jax.experimental import pallas as pl
from jax.experimental.pallas import tpu as pltpu
from jax.experimental.pallas import tpu_sc as plsc

N = 10000
E = 320000
D = 128
H = 128
NP = 10112

NC = 2
NS = 16
NW = NC * NS
CH = 128
NCHUNK = 80
WIN = 16
NWIN = NCHUNK // WIN
EPT = NCHUNK * CH
EPAD = NW * EPT
PAD_NODE = 10016
ROWS_PT = NP // NS

_sc_mesh = plsc.VectorSubcoreMesh(core_axis_name="c", subcore_axis_name="s")


def _zero_vmem(ref, nwords):
    z = jnp.zeros((16,), jnp.float32)

    def body(i, _):
        ref[pl.ds(i * 16, 16)] = z
        return 0

    lax.fori_loop(0, nwords // 16, body, 0)


@functools.partial(
    pl.kernel,
    out_type=jax.ShapeDtypeStruct((NC, 2, NP), jnp.float32),
    mesh=_sc_mesh,
    scratch_types=[
        pltpu.VMEM((NCHUNK, CH), jnp.int32),
        pltpu.VMEM((NCHUNK, CH), jnp.int32),
        pltpu.VMEM((CH,), jnp.float32),
        pltpu.VMEM((640,), jnp.float32),
        pltpu.VMEM_SHARED((NP,), jnp.float32),
        pltpu.VMEM_SHARED((NP,), jnp.float32),
    ],
)
def _degrees_sc(srcp_hbm, dstp_hbm, out_hbm, src_v, dst_v, ones_v, zer_v,
                hsrc_sh, hdst_sh):
    c = lax.axis_index("c")
    s = lax.axis_index("s")
    w = s * NC + c

    one = jnp.ones((16,), jnp.float32)

    def initb(i, _):
        ones_v[pl.ds(i * 16, 16)] = one
        return 0

    lax.fori_loop(0, CH // 16, initb, 0)
    _zero_vmem(zer_v, 640)

    pltpu.sync_copy(zer_v.at[pl.ds(0, ROWS_PT)],
                    hsrc_sh.at[pl.ds(s * ROWS_PT, ROWS_PT)])
    pltpu.sync_copy(zer_v.at[pl.ds(0, ROWS_PT)],
                    hdst_sh.at[pl.ds(s * ROWS_PT, ROWS_PT)])

    pltpu.sync_copy(srcp_hbm.at[w], src_v)
    pltpu.sync_copy(dstp_hbm.at[w], dst_v)

    plsc.subcore_barrier()

    def hist_body(j, _):
        pltpu.sync_copy(ones_v, hsrc_sh.at[src_v.at[j]], add=True)
        pltpu.sync_copy(ones_v, hdst_sh.at[dst_v.at[j]], add=True)
        return 0

    lax.fori_loop(0, NCHUNK, hist_body, 0)

    plsc.subcore_barrier()

    @pl.when(s == 0)
    def _():
        pltpu.sync_copy(hsrc_sh, out_hbm.at[c, 0, :])
        pltpu.sync_copy(hdst_sh, out_hbm.at[c, 1, :])


@functools.partial(
    pl.kernel,
    out_type=jax.ShapeDtypeStruct((NC, NP, H), jnp.float32),
    mesh=_sc_mesh,
    scratch_types=[
        pltpu.VMEM((WIN, CH), jnp.int32),
        pltpu.VMEM((WIN, CH), jnp.int32),
        pltpu.VMEM((WIN, CH), jnp.int32),
        pltpu.VMEM((WIN, CH), jnp.int32),
        pltpu.VMEM((CH, H), jnp.float32),
        pltpu.VMEM((CH, H), jnp.float32),
        pltpu.VMEM_SHARED((NP, H), jnp.float32),
        pltpu.SemaphoreType.DMA,
        pltpu.SemaphoreType.DMA,
        pltpu.SemaphoreType.DMA,
        pltpu.SemaphoreType.DMA,
        pltpu.SemaphoreType.DMA,
    ],
)
def _aggregate_sc(hs_hbm, srcp_hbm, dstp_hbm, out_hbm, swin0, swin1, dwin0,
                  dwin1, rows0, rows1, agg_sh, wsem, gsem0, gsem1, ssem0,
                  ssem1):
    c = lax.axis_index("c")
    s = lax.axis_index("s")
    w = s * NC + c
    swin = (swin0, swin1)
    dwin = (dwin0, dwin1)
    rows = (rows0, rows1)
    gsem = (gsem0, gsem1)
    ssem = (ssem0, ssem1)

    wd = [
        pltpu.async_copy(srcp_hbm.at[w, pl.ds(0, WIN), :], swin0, wsem),
        pltpu.async_copy(dstp_hbm.at[w, pl.ds(0, WIN), :], dwin0, wsem),
    ]

    z = jnp.zeros((16,), jnp.float32)

    def zrow(i, _):
        rows0[i // (H // 16), pl.ds((i % (H // 16)) * 16, 16)] = z
        return 0

    lax.fori_loop(0, CH * H // 16, zrow, 0)

    base = s * ROWS_PT

    def zbody(i, _):
        pltpu.sync_copy(rows0, agg_sh.at[pl.ds(base + i * CH, CH), :])
        return 0

    lax.fori_loop(0, ROWS_PT // CH, zbody, 0)
    rem = ROWS_PT % CH
    if rem:
        pltpu.sync_copy(
            rows0.at[pl.ds(0, rem), :],
            agg_sh.at[pl.ds(base + (ROWS_PT // CH) * CH, rem), :],
        )

    plsc.subcore_barrier()

    wdesc = {0: wd}
    gd = {}
    sprev = {}
    for p in range(NCHUNK + 1):
        if p < NCHUNK:
            g, r = p // WIN, p % WIN
            b = p % 2
            if r == 0:
                for d in wdesc[g]:
                    d.wait()
            if b in sprev:
                sprev.pop(b).wait()
            if r == 1 and g + 1 < NWIN:
                nb = (g + 1) % 2
                off = (g + 1) * WIN
                wdesc[g + 1] = [
                    pltpu.async_copy(srcp_hbm.at[w, pl.ds(off, WIN), :],
                                     swin[nb], wsem),
                    pltpu.async_copy(dstp_hbm.at[w, pl.ds(off, WIN), :],
                                     dwin[nb], wsem),
                ]
            gd[b] = (
                [
                    pltpu.async_copy(
                        hs_hbm.at[swin[g % 2].at[r, pl.ds(0, CH // 2)]],
                        rows[b].at[pl.ds(0, CH // 2), :], gsem[b]),
                    pltpu.async_copy(
                        hs_hbm.at[swin[g % 2].at[r, pl.ds(CH // 2, CH // 2)]],
                        rows[b].at[pl.ds(CH // 2, CH // 2), :], gsem[b]),
                ],
                dwin[g % 2],
                r,
            )
        ob = (p - 1) % 2
        if p >= 1:
            ds_, dwref, r_ = gd.pop(ob)
            for d in ds_:
                d.wait()
            sprev[ob] = pltpu.async_copy(rows[ob], agg_sh.at[dwref.at[r_]],
                                         ssem[ob], add=True)
    sprev[0].wait()
    sprev[1].wait()

    plsc.subcore_barrier()

    sl = pl.ds(s * ROWS_PT, ROWS_PT)
    pltpu.sync_copy(agg_sh.at[sl, :], out_hbm.at[c, sl, :])


def _norms_body(degp_ref, x_ref, hs_ref, ns_ref, nd_ref):
    d_src = degp_ref[0, 0] + degp_ref[1, 0]
    d_dst = degp_ref[0, 1] + degp_ref[1, 1]
    ns = jax.lax.rsqrt(jnp.maximum(d_src, 1.0))
    nd = jax.lax.rsqrt(jnp.maximum(d_dst, 1.0))
    ns_ref[...] = ns
    nd_ref[...] = nd
    hs_ref[...] = x_ref[...] * ns


def _norms_scale(degp, x_pad):
    RB = 1264
    grid = (NP // RB,)
    return pl.pallas_call(
        _norms_body,
        grid=grid,
        in_specs=[
            pl.BlockSpec((NC, 2, RB, 1), lambda i: (0, 0, i, 0)),
            pl.BlockSpec((RB, D), lambda i: (i, 0)),
        ],
        out_specs=[
            pl.BlockSpec((RB, D), lambda i: (i, 0)),
            pl.BlockSpec((RB, 1), lambda i: (i, 0)),
            pl.BlockSpec((RB, 1), lambda i: (i, 0)),
        ],
        out_shape=[
            jax.ShapeDtypeStruct((NP, D), jnp.float32),
            jax.ShapeDtypeStruct((NP, 1), jnp.float32),
            jax.ShapeDtypeStruct((NP, 1), jnp.float32),
        ],
    )(degp, x_pad)


def _layer_body(aggp_ref, ns_ref, nd_ref, w_ref, b_ref, out_ref, *, scale_out):
    a = (aggp_ref[0] + aggp_ref[1]) * nd_ref[...]
    h = jnp.dot(a, w_ref[...], preferred_element_type=jnp.float32) + b_ref[...]
    h = jnp.maximum(h, 0.0)
    if scale_out:
        h = h * ns_ref[...]
    out_ref[...] = h


def _layer_end(aggp, ns, nd, w, b, scale_out):
    RB = 1264
    grid = (NP // RB,)
    return pl.pallas_call(
        functools.partial(_layer_body, scale_out=scale_out),
        grid=grid,
        in_specs=[
            pl.BlockSpec((NC, RB, H), lambda i: (0, i, 0)),
            pl.BlockSpec((RB, 1), lambda i: (i, 0)),
            pl.BlockSpec((RB, 1), lambda i: (i, 0)),
            pl.BlockSpec((H, H), lambda i: (0, 0)),
            pl.BlockSpec((1, H), lambda i: (0, 0)),
        ],
        out_specs=pl.BlockSpec((RB, H), lambda i: (i, 0)),
        out_shape=jax.ShapeDtypeStruct((NP, H), jnp.float32),
    )(aggp, ns, nd, w, b.reshape(1, H))


def _head_body(xr_ref, wd_ref, bd_ref, out_ref):
    out_ref[...] = (
        jnp.dot(xr_ref[...], wd_ref[...], preferred_element_type=jnp.float32)
        + bd_ref[0, 0]
    )


def _head(xr, wd, bd):
    return pl.pallas_call(
        _head_body,
        out_shape=jax.ShapeDtypeStruct((N // 4, 1), jnp.float32),
    )(xr, wd, bd.reshape(1, 1))


def kernel(x, edge_index, W1, b1, W2, b2, Wd, bd):
    pad = (jnp.arange(EPAD - E, dtype=jnp.int32) % (NP - N)) + N
    srcp = jnp.concatenate([edge_index[0], pad]).reshape(NW, NCHUNK, CH)
    dstp = jnp.concatenate([edge_index[1], pad]).reshape(NW, NCHUNK, CH)
    x_pad = jnp.zeros((NP, D), x.dtype).at[:N].set(x)

    degp = _degrees_sc(srcp, dstp).reshape(NC, 2, NP, 1)

    hs1, ns, nd = _norms_scale(degp, x_pad)

    aggp1 = _aggregate_sc(hs1, srcp, dstp)
    hs2 = _layer_end(aggp1, ns, nd, W1, b1, scale_out=True)

    aggp2 = _aggregate_sc(hs2, srcp, dstp)
    h2 = _layer_end(aggp2, ns, nd, W2, b2, scale_out=False)

    xr = h2[:N].reshape(N // 4, 4 * H)
    return _head(xr, Wd, bd)


# speedup vs baseline: 1.2089x; 1.0369x over previous
import functools

import jax
import jax.numpy as jnp
from jax import lax
from jax.experimental import pallas as pl
from jax.experimental.pallas import tpu as pltpu
from jax.experimental.pallas import tpu_sc as plsc

N = 10000
E = 320000
D = 128
H = 128
NP = 10112

NC = 2
NS = 16
NW = NC * NS
CH = 128
NCHUNK = 80
WIN = 16
NWIN = NCHUNK // WIN
EPT = NCHUNK * CH
EPAD = NW * EPT
PAD_NODE = 10016
ROWS_PT = NP // NS

_sc_mesh = plsc.VectorSubcoreMesh(core_axis_name="c", subcore_axis_name="s")


def _zero_vmem(ref, nwords):
    z = jnp.zeros((16,), jnp.float32)

    def body(i, _):
        ref[pl.ds(i * 16, 16)] = z
        return 0

    lax.fori_loop(0, nwords // 16, body, 0)


@functools.partial(
    pl.kernel,
    out_type=jax.ShapeDtypeStruct((NC, 2, NP), jnp.float32),
    mesh=_sc_mesh,
    scratch_types=[
        pltpu.VMEM((NCHUNK, CH), jnp.int32),
        pltpu.VMEM((NCHUNK, CH), jnp.int32),
        pltpu.VMEM((CH,), jnp.float32),
        pltpu.VMEM((640,), jnp.float32),
        pltpu.VMEM_SHARED((NP,), jnp.float32),
        pltpu.VMEM_SHARED((NP,), jnp.float32),
    ],
)
def _degrees_sc(srcp_hbm, dstp_hbm, out_hbm, src_v, dst_v, ones_v, zer_v,
                hsrc_sh, hdst_sh):
    c = lax.axis_index("c")
    s = lax.axis_index("s")
    w = s * NC + c

    one = jnp.ones((16,), jnp.float32)

    def initb(i, _):
        ones_v[pl.ds(i * 16, 16)] = one
        return 0

    lax.fori_loop(0, CH // 16, initb, 0)
    _zero_vmem(zer_v, 640)

    pltpu.sync_copy(zer_v.at[pl.ds(0, ROWS_PT)],
                    hsrc_sh.at[pl.ds(s * ROWS_PT, ROWS_PT)])
    pltpu.sync_copy(zer_v.at[pl.ds(0, ROWS_PT)],
                    hdst_sh.at[pl.ds(s * ROWS_PT, ROWS_PT)])

    pltpu.sync_copy(srcp_hbm.at[w], src_v)
    pltpu.sync_copy(dstp_hbm.at[w], dst_v)

    plsc.subcore_barrier()

    def hist_body(j, _):
        pltpu.sync_copy(ones_v, hsrc_sh.at[src_v.at[j]], add=True)
        pltpu.sync_copy(ones_v, hdst_sh.at[dst_v.at[j]], add=True)
        return 0

    lax.fori_loop(0, NCHUNK, hist_body, 0)

    plsc.subcore_barrier()

    @pl.when(s == 0)
    def _():
        pltpu.sync_copy(hsrc_sh, out_hbm.at[c, 0, :])
        pltpu.sync_copy(hdst_sh, out_hbm.at[c, 1, :])


@functools.partial(
    pl.kernel,
    out_type=jax.ShapeDtypeStruct((NC, NP, H), jnp.float32),
    mesh=_sc_mesh,
    scratch_types=[
        pltpu.VMEM((WIN, CH), jnp.int32),
        pltpu.VMEM((WIN, CH), jnp.int32),
        pltpu.VMEM((WIN, CH), jnp.int32),
        pltpu.VMEM((WIN, CH), jnp.int32),
        pltpu.VMEM((CH, H), jnp.float32),
        pltpu.VMEM((CH, H), jnp.float32),
        pltpu.VMEM_SHARED((NP, H), jnp.float32),
        pltpu.SemaphoreType.DMA,
        pltpu.SemaphoreType.DMA,
        pltpu.SemaphoreType.DMA,
        pltpu.SemaphoreType.DMA,
        pltpu.SemaphoreType.DMA,
    ],
)
def _aggregate_sc(hs_hbm, srcp_hbm, dstp_hbm, out_hbm, swin0, swin1, dwin0,
                  dwin1, rows0, rows1, agg_sh, wsem, gsem0, gsem1, ssem0,
                  ssem1):
    c = lax.axis_index("c")
    s = lax.axis_index("s")
    w = s * NC + c
    swin = (swin0, swin1)
    dwin = (dwin0, dwin1)
    rows = (rows0, rows1)
    gsem = (gsem0, gsem1)
    ssem = (ssem0, ssem1)

    wd = [
        pltpu.async_copy(srcp_hbm.at[w, pl.ds(0, WIN), :], swin0, wsem),
        pltpu.async_copy(dstp_hbm.at[w, pl.ds(0, WIN), :], dwin0, wsem),
    ]

    z = jnp.zeros((16,), jnp.float32)

    def zrow(i, _):
        rows0[i // (H // 16), pl.ds((i % (H // 16)) * 16, 16)] = z
        return 0

    lax.fori_loop(0, CH * H // 16, zrow, 0)

    base = s * ROWS_PT

    def zbody(i, _):
        pltpu.sync_copy(rows0, agg_sh.at[pl.ds(base + i * CH, CH), :])
        return 0

    lax.fori_loop(0, ROWS_PT // CH, zbody, 0)
    rem = ROWS_PT % CH
    if rem:
        pltpu.sync_copy(
            rows0.at[pl.ds(0, rem), :],
            agg_sh.at[pl.ds(base + (ROWS_PT // CH) * CH, rem), :],
        )

    plsc.subcore_barrier()

    wdesc = {0: wd}
    gd = {}
    sprev = {}
    for p in range(NCHUNK + 1):
        if p < NCHUNK:
            g, r = p // WIN, p % WIN
            b = p % 2
            if r == 0:
                for d in wdesc[g]:
                    d.wait()
            if b in sprev:
                sprev.pop(b).wait()
            if r == 1 and g + 1 < NWIN:
                nb = (g + 1) % 2
                off = (g + 1) * WIN
                wdesc[g + 1] = [
                    pltpu.async_copy(srcp_hbm.at[w, pl.ds(off, WIN), :],
                                     swin[nb], wsem),
                    pltpu.async_copy(dstp_hbm.at[w, pl.ds(off, WIN), :],
                                     dwin[nb], wsem),
                ]
            gd[b] = (
                pltpu.async_copy(hs_hbm.at[swin[g % 2].at[r]], rows[b],
                                 gsem[b]),
                dwin[g % 2],
                r,
            )
        ob = (p - 1) % 2
        if p >= 1:
            d, dwref, r_ = gd.pop(ob)
            d.wait()
            sprev[ob] = pltpu.async_copy(rows[ob], agg_sh.at[dwref.at[r_]],
                                         ssem[ob], add=True)
    sprev[0].wait()
    sprev[1].wait()

    plsc.subcore_barrier()

    sl = pl.ds(s * ROWS_PT, ROWS_PT)
    pltpu.sync_copy(agg_sh.at[sl, :], out_hbm.at[c, sl, :])


def _norms_body(degp_ref, x_ref, hs_ref, ns_ref, nd_ref):
    d_src = degp_ref[:, 0:1] + degp_ref[:, 2:3]
    d_dst = degp_ref[:, 1:2] + degp_ref[:, 3:4]
    ns = jax.lax.rsqrt(jnp.maximum(d_src, 1.0))
    nd = jax.lax.rsqrt(jnp.maximum(d_dst, 1.0))
    ns_ref[...] = ns
    nd_ref[...] = nd
    hs_ref[...] = x_ref[...] * ns


def _norms_scale(degp, x_pad):
    RB = 1264
    grid = (NP // RB,)
    return pl.pallas_call(
        _norms_body,
        grid=grid,
        in_specs=[
            pl.BlockSpec((RB, 4), lambda i: (i, 0)),
            pl.BlockSpec((RB, D), lambda i: (i, 0)),
        ],
        out_specs=[
            pl.BlockSpec((RB, D), lambda i: (i, 0)),
            pl.BlockSpec((RB, 1), lambda i: (i, 0)),
            pl.BlockSpec((RB, 1), lambda i: (i, 0)),
        ],
        out_shape=[
            jax.ShapeDtypeStruct((NP, D), jnp.float32),
            jax.ShapeDtypeStruct((NP, 1), jnp.float32),
            jax.ShapeDtypeStruct((NP, 1), jnp.float32),
        ],
    )(degp, x_pad)


def _layer_body(aggp_ref, ns_ref, nd_ref, w_ref, b_ref, out_ref, *, scale_out):
    a = (aggp_ref[0] + aggp_ref[1]) * nd_ref[...]
    h = jnp.dot(a, w_ref[...], preferred_element_type=jnp.float32) + b_ref[...]
    h = jnp.maximum(h, 0.0)
    if scale_out:
        h = h * ns_ref[...]
    out_ref[...] = h


def _layer_end(aggp, ns, nd, w, b, scale_out):
    RB = 1264
    grid = (NP // RB,)
    return pl.pallas_call(
        functools.partial(_layer_body, scale_out=scale_out),
        grid=grid,
        in_specs=[
            pl.BlockSpec((NC, RB, H), lambda i: (0, i, 0)),
            pl.BlockSpec((RB, 1), lambda i: (i, 0)),
            pl.BlockSpec((RB, 1), lambda i: (i, 0)),
            pl.BlockSpec((H, H), lambda i: (0, 0)),
            pl.BlockSpec((1, H), lambda i: (0, 0)),
        ],
        out_specs=pl.BlockSpec((RB, H), lambda i: (i, 0)),
        out_shape=jax.ShapeDtypeStruct((NP, H), jnp.float32),
    )(aggp, ns, nd, w, b.reshape(1, H))


def _head_body(xr_ref, wd_ref, bd_ref, out_ref):
    out_ref[...] = (
        jnp.dot(xr_ref[...], wd_ref[...], preferred_element_type=jnp.float32)
        + bd_ref[0, 0]
    )


def _head(xr, wd, bd):
    return pl.pallas_call(
        _head_body,
        out_shape=jax.ShapeDtypeStruct((N // 4, 1), jnp.float32),
    )(xr, wd, bd.reshape(1, 1))


def kernel(x, edge_index, W1, b1, W2, b2, Wd, bd):
    pad = (jnp.arange(EPAD - E, dtype=jnp.int32) % (NP - N)) + N
    srcp = jnp.concatenate([edge_index[0], pad]).reshape(NW, NCHUNK, CH)
    dstp = jnp.concatenate([edge_index[1], pad]).reshape(NW, NCHUNK, CH)
    x_pad = jnp.zeros((NP, D), x.dtype).at[:N].set(x)

    degp = _degrees_sc(srcp, dstp).reshape(4, NP).T

    hs1, ns, nd = _norms_scale(degp, x_pad)

    aggp1 = _aggregate_sc(hs1, srcp, dstp)
    hs2 = _layer_end(aggp1, ns, nd, W1, b1, scale_out=True)

    aggp2 = _aggregate_sc(hs2, srcp, dstp)
    h2 = _layer_end(aggp2, ns, nd, W2, b2, scale_out=False)

    xr = h2[:N].reshape(N // 4, 4 * H)
    return _head(xr, Wd, bd)


# speedup vs baseline: 1.2344x; 1.0211x over previous
import functools

import jax
import jax.numpy as jnp
from jax import lax
from jax.experimental import pallas as pl
from jax.experimental.pallas import tpu as pltpu
from jax.experimental.pallas import tpu_sc as plsc

N = 10000
E = 320000
D = 128
H = 128
NP = 10112

NC = 2
NS = 16
NW = NC * NS
CH = 128
NCHUNK = 80
WIN = 16
NWIN = NCHUNK // WIN
EPT = NCHUNK * CH
EPAD = NW * EPT
PAD_NODE = 10016
ROWS_PT = NP // NS

_sc_mesh = plsc.VectorSubcoreMesh(core_axis_name="c", subcore_axis_name="s")


def _zero_vmem(ref, nwords):
    z = jnp.zeros((16,), jnp.float32)

    def body(i, _):
        ref[pl.ds(i * 16, 16)] = z
        return 0

    lax.fori_loop(0, nwords // 16, body, 0)


@functools.partial(
    pl.kernel,
    out_type=jax.ShapeDtypeStruct((NC, 2, NP), jnp.float32),
    mesh=_sc_mesh,
    scratch_types=[
        pltpu.VMEM((NCHUNK, CH), jnp.int32),
        pltpu.VMEM((NCHUNK, CH), jnp.int32),
        pltpu.VMEM((CH,), jnp.float32),
        pltpu.VMEM((640,), jnp.float32),
        pltpu.VMEM_SHARED((NP,), jnp.float32),
        pltpu.VMEM_SHARED((NP,), jnp.float32),
    ],
)
def _degrees_sc(srcp_hbm, dstp_hbm, out_hbm, src_v, dst_v, ones_v, zer_v,
                hsrc_sh, hdst_sh):
    c = lax.axis_index("c")
    s = lax.axis_index("s")
    w = s * NC + c

    one = jnp.ones((16,), jnp.float32)

    def initb(i, _):
        ones_v[pl.ds(i * 16, 16)] = one
        return 0

    lax.fori_loop(0, CH // 16, initb, 0)
    _zero_vmem(zer_v, 640)

    pltpu.sync_copy(zer_v.at[pl.ds(0, ROWS_PT)],
                    hsrc_sh.at[pl.ds(s * ROWS_PT, ROWS_PT)])
    pltpu.sync_copy(zer_v.at[pl.ds(0, ROWS_PT)],
                    hdst_sh.at[pl.ds(s * ROWS_PT, ROWS_PT)])

    pltpu.sync_copy(srcp_hbm.at[w], src_v)
    pltpu.sync_copy(dstp_hbm.at[w], dst_v)

    plsc.subcore_barrier()

    def hist_body(j, _):
        pltpu.sync_copy(ones_v, hsrc_sh.at[src_v.at[j]], add=True)
        pltpu.sync_copy(ones_v, hdst_sh.at[dst_v.at[j]], add=True)
        return 0

    lax.fori_loop(0, NCHUNK, hist_body, 0)

    plsc.subcore_barrier()

    @pl.when(s == 0)
    def _():
        pltpu.sync_copy(hsrc_sh, out_hbm.at[c, 0, :])
        pltpu.sync_copy(hdst_sh, out_hbm.at[c, 1, :])


@functools.partial(
    pl.kernel,
    out_type=jax.ShapeDtypeStruct((NC, NP, H), jnp.float32),
    mesh=_sc_mesh,
    scratch_types=[
        pltpu.VMEM((WIN, CH), jnp.int32),
        pltpu.VMEM((WIN, CH), jnp.int32),
        pltpu.VMEM((WIN, CH), jnp.int32),
        pltpu.VMEM((WIN, CH), jnp.int32),
        pltpu.VMEM((CH, H), jnp.float32),
        pltpu.VMEM((CH, H), jnp.float32),
        pltpu.VMEM_SHARED((NP, H), jnp.float32),
        pltpu.SemaphoreType.DMA,
        pltpu.SemaphoreType.DMA,
        pltpu.SemaphoreType.DMA,
        pltpu.SemaphoreType.DMA,
        pltpu.SemaphoreType.DMA,
    ],
)
def _aggregate_sc(hs_hbm, srcp_hbm, dstp_hbm, out_hbm, swin0, swin1, dwin0,
                  dwin1, rows0, rows1, agg_sh, wsem, gsem0, gsem1, ssem0,
                  ssem1):
    c = lax.axis_index("c")
    s = lax.axis_index("s")
    w = s * NC + c
    swin = (swin0, swin1)
    dwin = (dwin0, dwin1)
    rows = (rows0, rows1)
    gsem = (gsem0, gsem1)
    ssem = (ssem0, ssem1)

    wd = [
        pltpu.async_copy(srcp_hbm.at[w, pl.ds(0, WIN), :], swin0, wsem),
        pltpu.async_copy(dstp_hbm.at[w, pl.ds(0, WIN), :], dwin0, wsem),
    ]

    z = jnp.zeros((16,), jnp.float32)

    def zrow(i, _):
        rows0[i // (H // 16), pl.ds((i % (H // 16)) * 16, 16)] = z
        return 0

    lax.fori_loop(0, CH * H // 16, zrow, 0)

    base = s * ROWS_PT

    def zbody(i, _):
        pltpu.sync_copy(rows0, agg_sh.at[pl.ds(base + i * CH, CH), :])
        return 0

    lax.fori_loop(0, ROWS_PT // CH, zbody, 0)
    rem = ROWS_PT % CH
    if rem:
        pltpu.sync_copy(
            rows0.at[pl.ds(0, rem), :],
            agg_sh.at[pl.ds(base + (ROWS_PT // CH) * CH, rem), :],
        )

    plsc.subcore_barrier()

    wdesc = {0: wd}
    gd = {}
    sprev = {}
    for p in range(NCHUNK + 1):
        if p < NCHUNK:
            g, r = p // WIN, p % WIN
            b = p % 2
            if r == 0:
                for d in wdesc[g]:
                    d.wait()
            if b in sprev:
                sprev.pop(b).wait()
            if r == 1 and g + 1 < NWIN:
                nb = (g + 1) % 2
                off = (g + 1) * WIN
                wdesc[g + 1] = [
                    pltpu.async_copy(srcp_hbm.at[w, pl.ds(off, WIN), :],
                                     swin[nb], wsem),
                    pltpu.async_copy(dstp_hbm.at[w, pl.ds(off, WIN), :],
                                     dwin[nb], wsem),
                ]
            gd[b] = (
                pltpu.async_copy(hs_hbm.at[swin[g % 2].at[r]], rows[b],
                                 gsem[b]),
                dwin[g % 2],
                r,
            )
        ob = (p - 1) % 2
        if p >= 1:
            d, dwref, r_ = gd.pop(ob)
            d.wait()
            sprev[ob] = pltpu.async_copy(rows[ob], agg_sh.at[dwref.at[r_]],
                                         ssem[ob], add=True)
    sprev[0].wait()
    sprev[1].wait()

    plsc.subcore_barrier()

    sl = pl.ds(s * ROWS_PT, ROWS_PT)
    pltpu.sync_copy(agg_sh.at[sl, :], out_hbm.at[c, sl, :])


def _norms_body(degp_ref, x_ref, hs_ref, ns_ref, nd_ref):
    d_src = degp_ref[:, 0:1] + degp_ref[:, 2:3]
    d_dst = degp_ref[:, 1:2] + degp_ref[:, 3:4]
    ns = jax.lax.rsqrt(jnp.maximum(d_src, 1.0))
    nd = jax.lax.rsqrt(jnp.maximum(d_dst, 1.0))
    ns_ref[...] = ns
    nd_ref[...] = nd
    hs_ref[...] = x_ref[...] * ns


def _norms_scale(degp, x_pad):
    RB = 1264
    grid = (NP // RB,)
    return pl.pallas_call(
        _norms_body,
        grid=grid,
        in_specs=[
            pl.BlockSpec((RB, 4), lambda i: (i, 0)),
            pl.BlockSpec((RB, D), lambda i: (i, 0)),
        ],
        out_specs=[
            pl.BlockSpec((RB, D), lambda i: (i, 0)),
            pl.BlockSpec((RB, 1), lambda i: (i, 0)),
            pl.BlockSpec((RB, 1), lambda i: (i, 0)),
        ],
        out_shape=[
            jax.ShapeDtypeStruct((NP, D), jnp.float32),
            jax.ShapeDtypeStruct((NP, 1), jnp.float32),
            jax.ShapeDtypeStruct((NP, 1), jnp.float32),
        ],
    )(degp, x_pad)


def _layer_body(aggp_ref, ns_ref, nd_ref, w_ref, b_ref, out_ref, *, scale_out):
    a = (aggp_ref[0] + aggp_ref[1]) * nd_ref[...]
    h = jnp.dot(a, w_ref[...], preferred_element_type=jnp.float32) + b_ref[...]
    h = jnp.maximum(h, 0.0)
    if scale_out:
        h = h * ns_ref[...]
    out_ref[...] = h


RB2 = 2528


def _layer2_head_body(aggp_ref, nd_ref, w_ref, b_ref, wdm_ref, m_ref, g_ref,
                      bd_ref, out_ref):
    a = (aggp_ref[0] + aggp_ref[1]) * nd_ref[...]
    h = jnp.dot(a, w_ref[...], preferred_element_type=jnp.float32) + b_ref[...]
    h = jnp.maximum(h, 0.0)
    y = jnp.dot(h, wdm_ref[...], preferred_element_type=jnp.float32)
    srow = jnp.sum(y * m_ref[...], axis=1, keepdims=True)
    out_ref[...] = (
        jnp.dot(g_ref[...], srow, preferred_element_type=jnp.float32)
        + bd_ref[0, 0]
    )


def _layer2_head(aggp, nd, w, b, wd, bd):
    grid = (NP // RB2,)
    wdm = wd.reshape(4, H).T
    m4 = (jnp.arange(RB2)[:, None] % 4
          == jnp.arange(4)[None, :]).astype(jnp.float32)
    g4 = (jnp.arange(RB2)[None, :] // 4
          == jnp.arange(RB2 // 4)[:, None]).astype(jnp.float32)
    return pl.pallas_call(
        _layer2_head_body,
        grid=grid,
        in_specs=[
            pl.BlockSpec((NC, RB2, H), lambda i: (0, i, 0)),
            pl.BlockSpec((RB2, 1), lambda i: (i, 0)),
            pl.BlockSpec((H, H), lambda i: (0, 0)),
            pl.BlockSpec((1, H), lambda i: (0, 0)),
            pl.BlockSpec((H, 4), lambda i: (0, 0)),
            pl.BlockSpec((RB2, 4), lambda i: (0, 0)),
            pl.BlockSpec((RB2 // 4, RB2), lambda i: (0, 0)),
            pl.BlockSpec((1, 1), lambda i: (0, 0)),
        ],
        out_specs=pl.BlockSpec((RB2 // 4, 1), lambda i: (i, 0)),
        out_shape=jax.ShapeDtypeStruct((NP // 4, 1), jnp.float32),
    )(aggp, nd, w, b.reshape(1, H), wdm, m4, g4, bd.reshape(1, 1))


def _layer_end(aggp, ns, nd, w, b, scale_out):
    RB = 1264
    grid = (NP // RB,)
    return pl.pallas_call(
        functools.partial(_layer_body, scale_out=scale_out),
        grid=grid,
        in_specs=[
            pl.BlockSpec((NC, RB, H), lambda i: (0, i, 0)),
            pl.BlockSpec((RB, 1), lambda i: (i, 0)),
            pl.BlockSpec((RB, 1), lambda i: (i, 0)),
            pl.BlockSpec((H, H), lambda i: (0, 0)),
            pl.BlockSpec((1, H), lambda i: (0, 0)),
        ],
        out_specs=pl.BlockSpec((RB, H), lambda i: (i, 0)),
        out_shape=jax.ShapeDtypeStruct((NP, H), jnp.float32),
    )(aggp, ns, nd, w, b.reshape(1, H))


def _head_body(xr_ref, wd_ref, bd_ref, out_ref):
    out_ref[...] = (
        jnp.dot(xr_ref[...], wd_ref[...], preferred_element_type=jnp.float32)
        + bd_ref[0, 0]
    )


def _head(xr, wd, bd):
    return pl.pallas_call(
        _head_body,
        out_shape=jax.ShapeDtypeStruct((N // 4, 1), jnp.float32),
    )(xr, wd, bd.reshape(1, 1))


def kernel(x, edge_index, W1, b1, W2, b2, Wd, bd):
    pad = (jnp.arange(EPAD - E, dtype=jnp.int32) % (NP - N)) + N
    srcp = jnp.concatenate([edge_index[0], pad]).reshape(NW, NCHUNK, CH)
    dstp = jnp.concatenate([edge_index[1], pad]).reshape(NW, NCHUNK, CH)
    x_pad = jnp.zeros((NP, D), x.dtype).at[:N].set(x)

    degp = _degrees_sc(srcp, dstp).reshape(4, NP).T

    hs1, ns, nd = _norms_scale(degp, x_pad)

    aggp1 = _aggregate_sc(hs1, srcp, dstp)
    hs2 = _layer_end(aggp1, ns, nd, W1, b1, scale_out=True)

    aggp2 = _aggregate_sc(hs2, srcp, dstp)
    out4 = _layer2_head(aggp2, nd, W2, b2, Wd, bd)
    return out4[:N // 4]


# speedup vs baseline: 1.2356x; 1.0010x over previous
import functools

import jax
import jax.numpy as jnp
from jax import lax
from jax.experimental import pallas as pl
from jax.experimental.pallas import tpu as pltpu
from jax.experimental.pallas import tpu_sc as plsc

N = 10000
E = 320000
D = 128
H = 128
NP = 10112

NC = 2
NS = 16
NW = NC * NS
CH = 128
NCHUNK = 80
WIN = 16
NWIN = NCHUNK // WIN
EPT = NCHUNK * CH
EPAD = NW * EPT
PAD_NODE = 10016
ROWS_PT = NP // NS

_sc_mesh = plsc.VectorSubcoreMesh(core_axis_name="c", subcore_axis_name="s")


def _zero_vmem(ref, nwords):
    z = jnp.zeros((16,), jnp.float32)

    def body(i, _):
        ref[pl.ds(i * 16, 16)] = z
        return 0

    lax.fori_loop(0, nwords // 16, body, 0)


@functools.partial(
    pl.kernel,
    out_type=jax.ShapeDtypeStruct((NC, 2, NP), jnp.float32),
    mesh=_sc_mesh,
    scratch_types=[
        pltpu.VMEM((NCHUNK, CH), jnp.int32),
        pltpu.VMEM((NCHUNK, CH), jnp.int32),
        pltpu.VMEM((CH,), jnp.float32),
        pltpu.VMEM((640,), jnp.float32),
        pltpu.VMEM_SHARED((NP,), jnp.float32),
        pltpu.VMEM_SHARED((NP,), jnp.float32),
    ],
)
def _degrees_sc(srcp_hbm, dstp_hbm, out_hbm, src_v, dst_v, ones_v, zer_v,
                hsrc_sh, hdst_sh):
    c = lax.axis_index("c")
    s = lax.axis_index("s")
    w = s * NC + c

    one = jnp.ones((16,), jnp.float32)

    def initb(i, _):
        ones_v[pl.ds(i * 16, 16)] = one
        return 0

    lax.fori_loop(0, CH // 16, initb, 0)
    _zero_vmem(zer_v, 640)

    pltpu.sync_copy(zer_v.at[pl.ds(0, ROWS_PT)],
                    hsrc_sh.at[pl.ds(s * ROWS_PT, ROWS_PT)])
    pltpu.sync_copy(zer_v.at[pl.ds(0, ROWS_PT)],
                    hdst_sh.at[pl.ds(s * ROWS_PT, ROWS_PT)])

    pltpu.sync_copy(srcp_hbm.at[w], src_v)
    pltpu.sync_copy(dstp_hbm.at[w], dst_v)

    plsc.subcore_barrier()

    def hist_body(j, _):
        pltpu.sync_copy(ones_v, hsrc_sh.at[src_v.at[j]], add=True)
        pltpu.sync_copy(ones_v, hdst_sh.at[dst_v.at[j]], add=True)
        return 0

    lax.fori_loop(0, NCHUNK, hist_body, 0)

    plsc.subcore_barrier()

    @pl.when(s == 0)
    def _():
        pltpu.sync_copy(hsrc_sh, out_hbm.at[c, 0, :])
        pltpu.sync_copy(hdst_sh, out_hbm.at[c, 1, :])


@functools.partial(
    pl.kernel,
    out_type=jax.ShapeDtypeStruct((NC, NP, H), jnp.float32),
    mesh=_sc_mesh,
    scratch_types=[
        pltpu.VMEM((WIN, CH), jnp.int32),
        pltpu.VMEM((WIN, CH), jnp.int32),
        pltpu.VMEM((WIN, CH), jnp.int32),
        pltpu.VMEM((WIN, CH), jnp.int32),
        pltpu.VMEM((CH, H), jnp.float32),
        pltpu.VMEM((CH, H), jnp.float32),
        pltpu.VMEM_SHARED((NP, H), jnp.float32),
        pltpu.SemaphoreType.DMA,
        pltpu.SemaphoreType.DMA,
        pltpu.SemaphoreType.DMA,
        pltpu.SemaphoreType.DMA,
        pltpu.SemaphoreType.DMA,
    ],
)
def _aggregate_sc(hs_hbm, srcp_hbm, dstp_hbm, out_hbm, swin0, swin1, dwin0,
                  dwin1, rows0, rows1, agg_sh, wsem, gsem0, gsem1, ssem0,
                  ssem1):
    c = lax.axis_index("c")
    s = lax.axis_index("s")
    w = s * NC + c
    swin = (swin0, swin1)
    dwin = (dwin0, dwin1)
    rows = (rows0, rows1)
    gsem = (gsem0, gsem1)
    ssem = (ssem0, ssem1)

    wd = [
        pltpu.async_copy(srcp_hbm.at[w, pl.ds(0, WIN), :], swin0, wsem),
        pltpu.async_copy(dstp_hbm.at[w, pl.ds(0, WIN), :], dwin0, wsem),
    ]

    z = jnp.zeros((16,), jnp.float32)

    def zrow(i, _):
        rows0[i // (H // 16), pl.ds((i % (H // 16)) * 16, 16)] = z
        return 0

    lax.fori_loop(0, CH * H // 16, zrow, 0)

    base = s * ROWS_PT

    def zbody(i, _):
        pltpu.sync_copy(rows0, agg_sh.at[pl.ds(base + i * CH, CH), :])
        return 0

    lax.fori_loop(0, ROWS_PT // CH, zbody, 0)
    rem = ROWS_PT % CH
    if rem:
        pltpu.sync_copy(
            rows0.at[pl.ds(0, rem), :],
            agg_sh.at[pl.ds(base + (ROWS_PT // CH) * CH, rem), :],
        )

    plsc.subcore_barrier()

    wdesc = {0: wd}
    gd = {}
    sprev = {}
    for p in range(NCHUNK + 1):
        if p < NCHUNK:
            g, r = p // WIN, p % WIN
            b = p % 2
            if r == 0:
                for d in wdesc[g]:
                    d.wait()
            if b in sprev:
                sprev.pop(b).wait()
            if r == 1 and g + 1 < NWIN:
                nb = (g + 1) % 2
                off = (g + 1) * WIN
                wdesc[g + 1] = [
                    pltpu.async_copy(srcp_hbm.at[w, pl.ds(off, WIN), :],
                                     swin[nb], wsem),
                    pltpu.async_copy(dstp_hbm.at[w, pl.ds(off, WIN), :],
                                     dwin[nb], wsem),
                ]
            gd[b] = (
                pltpu.async_copy(hs_hbm.at[swin[g % 2].at[r]], rows[b],
                                 gsem[b]),
                dwin[g % 2],
                r,
            )
        ob = (p - 1) % 2
        if p >= 1:
            d, dwref, r_ = gd.pop(ob)
            d.wait()
            sprev[ob] = pltpu.async_copy(rows[ob], agg_sh.at[dwref.at[r_]],
                                         ssem[ob], add=True)
    sprev[0].wait()
    sprev[1].wait()

    plsc.subcore_barrier()

    sl = pl.ds(s * ROWS_PT, ROWS_PT)
    pltpu.sync_copy(agg_sh.at[sl, :], out_hbm.at[c, sl, :])


def _norms_body(degp_ref, x_ref, hs_ref, ns_ref, nd_ref):
    d_src = degp_ref[:, 0:1] + degp_ref[:, 2:3]
    d_dst = degp_ref[:, 1:2] + degp_ref[:, 3:4]
    ns = jax.lax.rsqrt(jnp.maximum(d_src, 1.0))
    nd = jax.lax.rsqrt(jnp.maximum(d_dst, 1.0))
    ns_ref[...] = ns
    nd_ref[...] = nd
    hs_ref[...] = x_ref[...] * ns


def _norms_scale(degp, x_pad):
    RB = 1264
    grid = (NP // RB,)
    return pl.pallas_call(
        _norms_body,
        grid=grid,
        in_specs=[
            pl.BlockSpec((RB, 4), lambda i: (i, 0)),
            pl.BlockSpec((RB, D), lambda i: (i, 0)),
        ],
        out_specs=[
            pl.BlockSpec((RB, D), lambda i: (i, 0)),
            pl.BlockSpec((RB, 1), lambda i: (i, 0)),
            pl.BlockSpec((RB, 1), lambda i: (i, 0)),
        ],
        out_shape=[
            jax.ShapeDtypeStruct((NP, D), jnp.float32),
            jax.ShapeDtypeStruct((NP, 1), jnp.float32),
            jax.ShapeDtypeStruct((NP, 1), jnp.float32),
        ],
    )(degp, x_pad)


def _layer_body(aggp_ref, ns_ref, nd_ref, w_ref, b_ref, out_ref, *, scale_out):
    a = (aggp_ref[0] + aggp_ref[1]) * nd_ref[...]
    h = jnp.dot(a, w_ref[...], preferred_element_type=jnp.float32) + b_ref[...]
    h = jnp.maximum(h, 0.0)
    if scale_out:
        h = h * ns_ref[...]
    out_ref[...] = h


RB2 = 2528


def _layer2_head_body(aggp_ref, nd_ref, w_ref, b_ref, wdm_ref, m_ref, g_ref,
                      bd_ref, out_ref):
    a = (aggp_ref[0] + aggp_ref[1]) * nd_ref[...]
    h = jnp.dot(a, w_ref[...], preferred_element_type=jnp.float32) + b_ref[...]
    h = jnp.maximum(h, 0.0)
    y = jnp.dot(h, wdm_ref[...], preferred_element_type=jnp.float32)
    srow = jnp.sum(y * m_ref[...], axis=1, keepdims=True)
    out_ref[...] = (
        jnp.dot(g_ref[...], srow, preferred_element_type=jnp.float32)
        + bd_ref[0, 0]
    )


def _layer2_head(aggp, nd, w, b, wd, bd):
    grid = (NP // RB2,)
    wdm = wd.reshape(4, H).T
    m4 = (jnp.arange(RB2)[:, None] % 4
          == jnp.arange(4)[None, :]).astype(jnp.float32)
    g4 = (jnp.arange(RB2)[None, :] // 4
          == jnp.arange(RB2 // 4)[:, None]).astype(jnp.float32)
    return pl.pallas_call(
        _layer2_head_body,
        grid=grid,
        in_specs=[
            pl.BlockSpec((NC, RB2, H), lambda i: (0, i, 0)),
            pl.BlockSpec((RB2, 1), lambda i: (i, 0)),
            pl.BlockSpec((H, H), lambda i: (0, 0)),
            pl.BlockSpec((1, H), lambda i: (0, 0)),
            pl.BlockSpec((H, 4), lambda i: (0, 0)),
            pl.BlockSpec((RB2, 4), lambda i: (0, 0)),
            pl.BlockSpec((RB2 // 4, RB2), lambda i: (0, 0)),
            pl.BlockSpec((1, 1), lambda i: (0, 0)),
        ],
        out_specs=pl.BlockSpec((RB2 // 4, 1), lambda i: (i, 0)),
        out_shape=jax.ShapeDtypeStruct((NP // 4, 1), jnp.float32),
    )(aggp, nd, w, b.reshape(1, H), wdm, m4, g4, bd.reshape(1, 1))


def _layer_end(aggp, ns, nd, w, b, scale_out):
    RB = 1264
    grid = (NP // RB,)
    return pl.pallas_call(
        functools.partial(_layer_body, scale_out=scale_out),
        grid=grid,
        in_specs=[
            pl.BlockSpec((NC, RB, H), lambda i: (0, i, 0)),
            pl.BlockSpec((RB, 1), lambda i: (i, 0)),
            pl.BlockSpec((RB, 1), lambda i: (i, 0)),
            pl.BlockSpec((H, H), lambda i: (0, 0)),
            pl.BlockSpec((1, H), lambda i: (0, 0)),
        ],
        out_specs=pl.BlockSpec((RB, H), lambda i: (i, 0)),
        out_shape=jax.ShapeDtypeStruct((NP, H), jnp.float32),
    )(aggp, ns, nd, w, b.reshape(1, H))


def kernel(x, edge_index, W1, b1, W2, b2, Wd, bd):
    pad = (jnp.arange(EPAD - E, dtype=jnp.int32) % (NP - N)) + N
    srcp = jnp.concatenate([edge_index[0], pad]).reshape(NW, NCHUNK, CH)
    dstp = jnp.concatenate([edge_index[1], pad]).reshape(NW, NCHUNK, CH)
    x_pad = jnp.zeros((NP, D), x.dtype).at[:N].set(x)

    degp = _degrees_sc(srcp, dstp).reshape(4, NP).T

    hs1, ns, nd = _norms_scale(degp, x_pad)

    aggp1 = _aggregate_sc(hs1, srcp, dstp)
    hs2 = _layer_end(aggp1, ns, nd, W1, b1, scale_out=True)

    aggp2 = _aggregate_sc(hs2, srcp, dstp)
    out4 = _layer2_head(aggp2, nd, W2, b2, Wd, bd)
    return out4[:N // 4]
